# baseline (speedup 1.0000x reference)
import functools

import jax
import jax.numpy as jnp
from jax import lax
from jax.experimental import pallas as pl
from jax.experimental.pallas import tpu as pltpu
from jax.experimental.pallas import tpu_sc as plsc

_IDX_CHUNK = 128


@functools.lru_cache(maxsize=None)
def _build(V, B, fs, w, h):
    info = plsc.get_sparse_core_info()
    NC, NS = info.num_cores, info.num_subcores
    NW = NC * NS
    bpw = B // NW
    nchunk = bpw // _IDX_CHUNK
    echunk = 4 * nchunk
    mesh = plsc.VectorSubcoreMesh(core_axis_name="c", subcore_axis_name="s")

    @functools.partial(
        pl.kernel,
        mesh=mesh,
        compiler_params=pltpu.CompilerParams(needs_layout_passes=False),
        out_type=(
            jax.ShapeDtypeStruct((B * 9,), jnp.float32),
            jax.ShapeDtypeStruct((B,), jnp.float32),
        ),
        scratch_types=[
            pltpu.VMEM((nchunk, _IDX_CHUNK), jnp.int32),
            pltpu.VMEM((echunk, _IDX_CHUNK), jnp.int32),
            pltpu.VMEM((bpw * 4,), jnp.float32),
            pltpu.VMEM((bpw,), jnp.float32),
            pltpu.VMEM((bpw * 9,), jnp.float32),
            pltpu.SemaphoreType.DMA,
            pltpu.SemaphoreType.DMA,
        ],
    )
    def k(idx_hbm, eidx_hbm, tab_hbm, dist_hbm, pat_hbm, out9_hbm, dout_hbm,
          idx_v, eidx_v, rows_f, dvm, out_v, sem_r, sem_d):
        wid = lax.axis_index("s") * NC + lax.axis_index("c")
        pltpu.sync_copy(idx_hbm.at[pl.ds(wid * nchunk, nchunk)], idx_v)
        pltpu.sync_copy(eidx_hbm.at[pl.ds(wid * echunk, echunk)], eidx_v)
        rcopies = []
        dcopies = []
        for j in range(echunk):
            rcopies.append(pltpu.async_copy(
                tab_hbm.at[eidx_v.at[j]],
                rows_f.at[pl.ds(j * _IDX_CHUNK, _IDX_CHUNK)], sem_r))
        for j in range(nchunk):
            dcopies.append(pltpu.async_copy(
                dist_hbm.at[idx_v.at[j]],
                dvm.at[pl.ds(j * _IDX_CHUNK, _IDX_CHUNK)], sem_d))
        pltpu.sync_copy(pat_hbm, out_v)
        for c in rcopies:
            c.wait()

        iota = lax.iota(jnp.int32, 16)
        q = lax.shift_right_logical(iota, 2)
        comp = jnp.bitwise_and(iota, 3)
        m = jnp.where(comp == 1, 4,
                      jnp.where(comp == 2, 2,
                                jnp.where(comp == 3, 5, 0)))
        base_idx = 9 * q + m
        s = jnp.where(comp < 2, jnp.float32(fs),
                      jnp.where(comp == 2, jnp.float32(w), jnp.float32(h)))

        def step(i, carry):
            v = rows_f[pl.ds(16 * i, 16)]
            plsc.store_scatter(out_v, [36 * i + base_idx], v * s)
            return carry

        lax.fori_loop(0, bpw // 4, step, 0)

        pltpu.sync_copy(out_v, out9_hbm.at[pl.ds(wid * bpw * 9, bpw * 9)])
        for c in dcopies:
            c.wait()
        pltpu.sync_copy(dvm, dout_hbm.at[pl.ds(wid * bpw, bpw)])

    return k


def kernel(input, video_idx, intrinsics_factors, distortion):
    H, W = input.shape[1], input.shape[2]
    fs = 0.5 * (H + W)
    V = intrinsics_factors.shape[0]
    B = video_idx.shape[0]
    k = _build(V, B, float(fs), float(W), float(H))
    pattern = jnp.tile(
        jnp.array([0.0] * 8 + [1.0], dtype=jnp.float32), B // 32)
    idx32 = video_idx.astype(jnp.int32)
    eidx = (idx32[:, None] * 4 + jnp.arange(4, dtype=jnp.int32)[None, :])
    out9, dout = k(
        idx32.reshape(-1, _IDX_CHUNK),
        eidx.reshape(-1, _IDX_CHUNK),
        intrinsics_factors.reshape(-1),
        distortion,
        pattern,
    )
    return out9.reshape(B, 3, 3), dout.reshape(B, 1, 1)


# speedup vs baseline: 17.3977x; 17.3977x over previous
import functools

import jax
import jax.numpy as jnp
from jax import lax
from jax.experimental import pallas as pl
from jax.experimental.pallas import tpu as pltpu
from jax.experimental.pallas import tpu_sc as plsc

_IDX_CHUNK = 128


@functools.lru_cache(maxsize=None)
def _build(V, B, fs, w, h):
    info = plsc.get_sparse_core_info()
    NC, NS = info.num_cores, info.num_subcores
    NW = NC * NS
    bpw = B // NW
    nchunk = bpw // _IDX_CHUNK
    mesh = plsc.VectorSubcoreMesh(core_axis_name="c", subcore_axis_name="s")
    scale = (fs, fs, w, h)

    @functools.partial(
        pl.kernel,
        mesh=mesh,
        compiler_params=pltpu.CompilerParams(needs_layout_passes=False),
        out_type=tuple(
            jax.ShapeDtypeStruct((B,), jnp.float32) for _ in range(5)),
        scratch_types=[
            pltpu.VMEM((nchunk, _IDX_CHUNK), jnp.int32),
        ] + [pltpu.VMEM((bpw,), jnp.float32) for _ in range(5)] + [
            pltpu.SemaphoreType.DMA,
        ],
    )
    def k(idx_hbm, c0_hbm, c1_hbm, c2_hbm, c3_hbm, dist_hbm,
          fx_hbm, fy_hbm, x0_hbm, y0_hbm, dout_hbm,
          idx_v, g0, g1, g2, g3, g4, sem):
        wid = lax.axis_index("s") * NC + lax.axis_index("c")
        pltpu.sync_copy(idx_hbm.at[pl.ds(wid * nchunk, nchunk)], idx_v)
        bufs = (g0, g1, g2, g3, g4)
        copies = []
        srcs = (c0_hbm, c1_hbm, c2_hbm, c3_hbm, dist_hbm)
        for ci, col_hbm in enumerate(srcs):
            for j in range(nchunk):
                copies.append(pltpu.async_copy(
                    col_hbm.at[idx_v.at[j]],
                    bufs[ci].at[pl.ds(j * _IDX_CHUNK, _IDX_CHUNK)], sem))
        for c in copies:
            c.wait()

        def step(i, carry):
            for ci in range(4):
                sl = pl.ds(16 * i, 16)
                bufs[ci][sl] = bufs[ci][sl] * jnp.float32(scale[ci])
            return carry

        lax.fori_loop(0, bpw // 16, step, 0)

        dsts = (fx_hbm, fy_hbm, x0_hbm, y0_hbm, dout_hbm)
        for ci, dst_hbm in enumerate(dsts):
            pltpu.sync_copy(bufs[ci], dst_hbm.at[pl.ds(wid * bpw, bpw)])

    return k


def kernel(input, video_idx, intrinsics_factors, distortion):
    H, W = input.shape[1], input.shape[2]
    fs = 0.5 * (H + W)
    V = intrinsics_factors.shape[0]
    B = video_idx.shape[0]
    k = _build(V, B, float(fs), float(W), float(H))
    idx32 = video_idx.astype(jnp.int32)
    fx, fy, x0, y0, dist = k(
        idx32.reshape(-1, _IDX_CHUNK),
        intrinsics_factors[:, 0],
        intrinsics_factors[:, 1],
        intrinsics_factors[:, 2],
        intrinsics_factors[:, 3],
        distortion,
    )
    zero = jnp.zeros_like(fx)
    one = jnp.ones_like(fx)
    row0 = jnp.stack([fx, zero, x0], axis=-1)
    row1 = jnp.stack([zero, fy, y0], axis=-1)
    row2 = jnp.stack([zero, zero, one], axis=-1)
    int_mat = jnp.stack([row0, row1, row2], axis=1)
    return int_mat, dist.reshape(B, 1, 1)


# speedup vs baseline: 19.0331x; 1.0940x over previous
import functools

import jax
import jax.numpy as jnp
from jax import lax
from jax.experimental import pallas as pl
from jax.experimental.pallas import tpu as pltpu
from jax.experimental.pallas import tpu_sc as plsc

_IDX_CHUNK = 128
_TILE = 128


@functools.lru_cache(maxsize=None)
def _build(V, B, fs, w, h):
    info = plsc.get_sparse_core_info()
    NC, NS = info.num_cores, info.num_subcores
    NW = NC * NS
    bpw = B // NW
    nchunk = bpw // _IDX_CHUNK
    vpad = (V + _TILE - 1) // _TILE * _TILE
    mesh = plsc.VectorSubcoreMesh(core_axis_name="c", subcore_axis_name="s")
    scale = (fs, fs, w, h)

    @functools.partial(
        pl.kernel,
        mesh=mesh,
        compiler_params=pltpu.CompilerParams(needs_layout_passes=False),
        out_type=tuple(
            jax.ShapeDtypeStruct((B,), jnp.float32) for _ in range(5)),
        scratch_types=[
            pltpu.VMEM((nchunk, _IDX_CHUNK), jnp.int32),
        ] + [pltpu.VMEM((nchunk, _IDX_CHUNK), jnp.int32) for _ in range(4)
        ] + [pltpu.VMEM((bpw,), jnp.float32) for _ in range(5)] + [
            pltpu.SemaphoreType.DMA,
            pltpu.SemaphoreType.DMA,
        ],
    )
    def k(idx_hbm, flat_hbm, dist_hbm,
          fx_hbm, fy_hbm, x0_hbm, y0_hbm, dout_hbm,
          idx_v, a0, a1, a2, a3, g0, g1, g2, g3, g4, sem_c, sem_d):
        wid = lax.axis_index("s") * NC + lax.axis_index("c")
        pltpu.sync_copy(idx_hbm.at[pl.ds(wid * nchunk, nchunk)], idx_v)
        abufs = (a0, a1, a2, a3)
        gbufs = (g0, g1, g2, g3, g4)

        dcopies = [
            pltpu.async_copy(
                dist_hbm.at[idx_v.at[j]],
                g4.at[pl.ds(j * _IDX_CHUNK, _IDX_CHUNK)], sem_d)
            for j in range(nchunk)
        ]

        def addr_step(t, carry):
            j = t // 8
            sl = pl.ds(16 * (t % 8), 16)
            rv = idx_v[j, sl]
            for ci in range(4):
                abufs[ci][j, sl] = rv + ci * vpad
            return carry

        lax.fori_loop(0, nchunk * 8, addr_step, 0)

        ccopies = []
        for ci in range(4):
            for j in range(nchunk):
                ccopies.append(pltpu.async_copy(
                    flat_hbm.at[abufs[ci].at[j]],
                    gbufs[ci].at[pl.ds(j * _IDX_CHUNK, _IDX_CHUNK)], sem_c))
        for c in ccopies:
            c.wait()

        def scale_step(i, carry):
            sl = pl.ds(16 * i, 16)
            for ci in range(4):
                gbufs[ci][sl] = gbufs[ci][sl] * jnp.float32(scale[ci])
            return carry

        lax.fori_loop(0, bpw // 16, scale_step, 0)

        for c in dcopies:
            c.wait()
        dsts = (fx_hbm, fy_hbm, x0_hbm, y0_hbm, dout_hbm)
        for ci, dst_hbm in enumerate(dsts):
            pltpu.sync_copy(gbufs[ci], dst_hbm.at[pl.ds(wid * bpw, bpw)])

    return k


def kernel(input, video_idx, intrinsics_factors, distortion):
    H, W = input.shape[1], input.shape[2]
    fs = 0.5 * (H + W)
    V = intrinsics_factors.shape[0]
    B = video_idx.shape[0]
    k = _build(V, B, float(fs), float(W), float(H))
    idx32 = video_idx.astype(jnp.int32)
    nt = (V + _TILE - 1) // _TILE
    flat = jnp.pad(
        intrinsics_factors, ((0, nt * _TILE - V), (0, 0))).T.reshape(-1)
    fx, fy, x0, y0, dist = k(
        idx32.reshape(-1, _IDX_CHUNK),
        flat,
        distortion,
    )
    zero = jnp.zeros_like(fx)
    one = jnp.ones_like(fx)
    row0 = jnp.stack([fx, zero, x0], axis=-1)
    row1 = jnp.stack([zero, fy, y0], axis=-1)
    row2 = jnp.stack([zero, zero, one], axis=-1)
    int_mat = jnp.stack([row0, row1, row2], axis=1)
    return int_mat, dist.reshape(B, 1, 1)


# speedup vs baseline: 28.3829x; 1.4912x over previous
import functools

import jax
import jax.numpy as jnp
from jax import lax
from jax.experimental import pallas as pl
from jax.experimental.pallas import tpu as pltpu
from jax.experimental.pallas import tpu_sc as plsc

_IDX_CHUNK = 128
_TILE = 128


@functools.lru_cache(maxsize=None)
def _build(V, B, fs, w, h):
    info = plsc.get_sparse_core_info()
    NC, NS = info.num_cores, info.num_subcores
    NW = NC * NS
    bpw = B // NW
    nchunk = bpw // _IDX_CHUNK
    mesh = plsc.VectorSubcoreMesh(core_axis_name="c", subcore_axis_name="s")
    scale = (fs, fs, w, h)

    @functools.partial(
        pl.kernel,
        mesh=mesh,
        compiler_params=pltpu.CompilerParams(needs_layout_passes=False),
        out_type=tuple(
            jax.ShapeDtypeStruct((B,), jnp.float32) for _ in range(5)),
        scratch_types=[
            pltpu.VMEM((nchunk, _IDX_CHUNK), jnp.int32),
            pltpu.VMEM((nchunk, _IDX_CHUNK), jnp.int32),
            pltpu.VMEM((_IDX_CHUNK, 4, _TILE), jnp.float32),
        ] + [pltpu.VMEM((bpw,), jnp.float32) for _ in range(5)] + [
            pltpu.SemaphoreType.DMA,
            pltpu.SemaphoreType.DMA,
        ],
    )
    def k(idx_hbm, tab3_hbm, dist_hbm,
          fx_hbm, fy_hbm, x0_hbm, y0_hbm, dout_hbm,
          idx_v, tidx_v, blk_v, g0, g1, g2, g3, g4, sem_c, sem_d):
        wid = lax.axis_index("s") * NC + lax.axis_index("c")
        pltpu.sync_copy(idx_hbm.at[pl.ds(wid * nchunk, nchunk)], idx_v)
        gbufs = (g0, g1, g2, g3)

        dcopies = [
            pltpu.async_copy(
                dist_hbm.at[idx_v.at[j]],
                g4.at[pl.ds(j * _IDX_CHUNK, _IDX_CHUNK)], sem_d)
            for j in range(nchunk)
        ]

        def tile_step(t, carry):
            j = t // 8
            sl = pl.ds(16 * (t % 8), 16)
            tidx_v[j, sl] = lax.shift_right_logical(idx_v[j, sl], 7)
            return carry

        lax.fori_loop(0, nchunk * 8, tile_step, 0)

        iota = lax.iota(jnp.int32, 16)

        for j in range(nchunk):
            pltpu.async_copy(
                tab3_hbm.at[tidx_v.at[j]], blk_v, sem_c).wait()

            def extract_step(g, carry):
                jv = 16 * g + iota
                rv = idx_v[j, pl.ds(16 * g, 16)]
                iv = rv & 127
                for ci in range(4):
                    v = plsc.load_gather(blk_v, [jv, iota * 0 + ci, iv])
                    gbufs[ci][pl.ds(j * _IDX_CHUNK + 16 * g, 16)] = (
                        v * jnp.float32(scale[ci]))
                return carry

            lax.fori_loop(0, _IDX_CHUNK // 16, extract_step, 0)

        for c in dcopies:
            c.wait()
        dsts = (fx_hbm, fy_hbm, x0_hbm, y0_hbm, dout_hbm)
        allbufs = (g0, g1, g2, g3, g4)
        for ci, dst_hbm in enumerate(dsts):
            pltpu.sync_copy(allbufs[ci], dst_hbm.at[pl.ds(wid * bpw, bpw)])

    return k


def kernel(input, video_idx, intrinsics_factors, distortion):
    H, W = input.shape[1], input.shape[2]
    fs = 0.5 * (H + W)
    V = intrinsics_factors.shape[0]
    B = video_idx.shape[0]
    k = _build(V, B, float(fs), float(W), float(H))
    idx32 = video_idx.astype(jnp.int32)
    nt = (V + _TILE - 1) // _TILE
    padded = jnp.pad(intrinsics_factors, ((0, nt * _TILE - V), (0, 0)))
    view3 = padded.reshape(nt, _TILE, 4).transpose(0, 2, 1)
    fx, fy, x0, y0, dist = k(
        idx32.reshape(-1, _IDX_CHUNK),
        view3,
        distortion,
    )
    zero = jnp.zeros_like(fx)
    one = jnp.ones_like(fx)
    row0 = jnp.stack([fx, zero, x0], axis=-1)
    row1 = jnp.stack([zero, fy, y0], axis=-1)
    row2 = jnp.stack([zero, zero, one], axis=-1)
    int_mat = jnp.stack([row0, row1, row2], axis=1)
    return int_mat, dist.reshape(B, 1, 1)
